# baseline (speedup 1.0000x reference)
import functools
import jax
import jax.numpy as jnp
from jax import lax
from jax.experimental import pallas as pl
from jax.experimental.pallas import tpu as pltpu
from jax.experimental.pallas import tpu_sc as plsc

_B = 1024
_K = 100000
_D = 128
_T = 0.07
_INV_T = 1.0 / _T

_NC = 2
_NS = 16
_NW = _NC * _NS
_BPW = _B // _NW
_SLAB = 3128
_SLAB_LAST = _K - _SLAB * (_NW - 1)
_PAD = _B + _NW * 16

_WK = 2048
_NKT = (_K + _WK - 1) // _WK


def _sc_gather_body(ml_hbm, mab_hbm, y_hbm, gl_hbm, gab_hbm, idx_v, rows_v, sem):
    wid = lax.axis_index("s") * _NC + lax.axis_index("c")
    base = wid * _BPW
    pltpu.sync_copy(y_hbm.at[pl.ds(base, _BPW)], idx_v)
    pltpu.async_copy(ml_hbm.at[idx_v], rows_v, sem).wait()
    pltpu.sync_copy(rows_v, gl_hbm.at[pl.ds(base, _BPW)])
    pltpu.async_copy(mab_hbm.at[idx_v], rows_v, sem).wait()
    pltpu.sync_copy(rows_v, gab_hbm.at[pl.ds(base, _BPW)])


_sc_gather = pl.kernel(
    _sc_gather_body,
    out_type=(jax.ShapeDtypeStruct((_B, _D), jnp.float32),
              jax.ShapeDtypeStruct((_B, _D), jnp.float32)),
    mesh=plsc.VectorSubcoreMesh(core_axis_name="c", subcore_axis_name="s",
                                num_cores=_NC, num_subcores=_NS),
    scratch_types=[pltpu.VMEM((_BPW,), jnp.int32),
                   pltpu.VMEM((_BPW, _D), jnp.float32),
                   pltpu.SemaphoreType.DMA],
    compiler_params=pltpu.CompilerParams(needs_layout_passes=False),
)


def _prep_body(l_ref, ab_ref, gl_ref, gab_ref, yr_ref, yc_ref, ml0_ref, mab0_ref,
               posl_ref, posab_ref, sv_ref, ylp_ref, startp_ref, count_ref):
    yr = yr_ref[...]
    yc = yc_ref[...]
    eq = yc == yr
    rowb = lax.broadcasted_iota(jnp.int32, (_B, _B), 0)
    colb = lax.broadcasted_iota(jnp.int32, (_B, _B), 1)
    winner = jnp.max(jnp.where(eq, colb, -1), axis=1, keepdims=True)
    onehot = (colb == winner).astype(jnp.float32)

    hp = lax.Precision.HIGHEST
    lw = jnp.dot(onehot, l_ref[...], precision=hp)
    abw = jnp.dot(onehot, ab_ref[...], precision=hp)
    glw = jnp.dot(onehot, gl_ref[...], precision=hp)
    gabw = jnp.dot(onehot, gab_ref[...], precision=hp)

    posl = glw * 0.5 + lw * 0.5
    posl = posl / jnp.sqrt(jnp.sum(posl * posl, axis=1, keepdims=True))
    posab = gabw * 0.5 + abw * 0.5
    posab = posab / jnp.sqrt(jnp.sum(posab * posab, axis=1, keepdims=True))

    v_y_ab = jnp.sum(ab_ref[...] * gl_ref[...], axis=1, keepdims=True)
    v_y_l = jnp.sum(l_ref[...] * gab_ref[...], axis=1, keepdims=True)
    v_0_ab = jnp.sum(ab_ref[...] * ml0_ref[...], axis=1, keepdims=True)
    v_0_l = jnp.sum(l_ref[...] * mab0_ref[...], axis=1, keepdims=True)
    sv_ref[...] = jnp.concatenate(
        [v_y_ab, v_y_l, v_0_ab, v_0_l, v_y_ab, v_y_l, v_0_ab, v_0_l], axis=1)

    tbound_row = lax.broadcasted_iota(jnp.int32, (1, _NW), 1) * _SLAB
    tbound_col = lax.broadcasted_iota(jnp.int32, (_NW, 1), 0) * _SLAB
    tile_col = jnp.sum((yc >= tbound_row).astype(jnp.int32),
                       axis=1, keepdims=True) - 1
    tile_row = jnp.sum((yr >= tbound_col).astype(jnp.int32),
                       axis=0, keepdims=True) - 1
    teq = tile_col == tile_row
    rank = jnp.sum(jnp.where(teq & (colb < rowb), 1, 0),
                   axis=1, keepdims=True)

    onehot_bt = (tile_col == lax.broadcasted_iota(jnp.int32, (1, _NW), 1))
    onehot_bt = onehot_bt.astype(jnp.float32)
    dn0 = (((0,), (0,)), ((), ()))
    ones_b = jnp.ones((_B, 1), jnp.float32)
    count = lax.dot_general(onehot_bt, ones_b, dn0, precision=hp)
    count_i = count.astype(jnp.int32)
    countp_i = lax.shift_left(lax.shift_right_logical(count_i + 15, 2 + 2), 4)
    countp = countp_i.astype(jnp.float32)
    lt32 = (lax.broadcasted_iota(jnp.int32, (_NW, _NW), 1)
            < lax.broadcasted_iota(jnp.int32, (_NW, _NW), 0)).astype(jnp.float32)
    startp = jnp.dot(lt32, countp, precision=hp)

    startp_b = jnp.dot(onehot_bt, startp, precision=hp).astype(jnp.int32)
    count_b = jnp.dot(onehot_bt, count, precision=hp).astype(jnp.int32)
    countp_b = jnp.dot(onehot_bt, countp, precision=hp).astype(jnp.int32)
    posidx = startp_b + rank

    p_row = lax.broadcasted_iota(jnp.int32, (_B, _PAD), 1)
    padmask = ((rank == 0) & (p_row >= startp_b + count_b)
               & (p_row < startp_b + countp_b))
    pt = ((p_row == posidx) | padmask).astype(jnp.float32)

    posl_ref[...] = lax.dot_general(pt, posl, dn0, precision=hp)
    posab_ref[...] = lax.dot_general(pt, posab, dn0, precision=hp)
    ylp_ref[...] = lax.dot_general(pt, yc.astype(jnp.float32), dn0,
                                   precision=hp).astype(jnp.int32)
    startp_ref[...] = startp.astype(jnp.int32)
    count_ref[...] = count_i


def _tc_prep(l, ab, gl, gab, y, ml0, mab0):
    return pl.pallas_call(
        _prep_body,
        out_shape=(jax.ShapeDtypeStruct((_PAD, _D), jnp.float32),
                   jax.ShapeDtypeStruct((_PAD, _D), jnp.float32),
                   jax.ShapeDtypeStruct((_B, 8), jnp.float32),
                   jax.ShapeDtypeStruct((_PAD, 1), jnp.int32),
                   jax.ShapeDtypeStruct((_NW, 1), jnp.int32),
                   jax.ShapeDtypeStruct((_NW, 1), jnp.int32)),
    )(l, ab, gl, gab, y.reshape(1, _B), y.reshape(_B, 1), ml0, mab0)


def _z_body(ml_ref, mab_ref, l_ref, ab_ref, out_ref):
    i = pl.program_id(0)

    @pl.when(i == 0)
    def _init():
        out_ref[0] = 0.0
        out_ref[1] = 0.0

    col = lax.broadcasted_iota(jnp.int32, (_B, _WK), 1) + i * _WK
    valid = col < _K
    dn = (((1,), (1,)), ((), ()))
    s_ab = lax.dot_general(ab_ref[...], ml_ref[...], dn) * _INV_T
    s_l = lax.dot_general(l_ref[...], mab_ref[...], dn) * _INV_T
    out_ref[0] += jnp.sum(jnp.where(valid, jnp.exp(s_ab), 0.0))
    out_ref[1] += jnp.sum(jnp.where(valid, jnp.exp(s_l), 0.0))


def _tc_z(memory_l, memory_ab, l, ab):
    return pl.pallas_call(
        _z_body,
        grid=(_NKT,),
        in_specs=[
            pl.BlockSpec((_WK, _D), lambda i: (i, 0)),
            pl.BlockSpec((_WK, _D), lambda i: (i, 0)),
            pl.BlockSpec((_B, _D), lambda i: (0, 0)),
            pl.BlockSpec((_B, _D), lambda i: (0, 0)),
        ],
        out_specs=pl.BlockSpec(memory_space=pltpu.SMEM),
        out_shape=jax.ShapeDtypeStruct((2,), jnp.float32),
    )(memory_l, memory_ab, l, ab)


_NKT1 = _NKT // 2
_KH1 = _NKT1 * _WK
_KH2 = _K - _KH1


def _make_main_body(cy, c0, kbase):
    def _main_body(m_ref, f_ref, yc_ref, sv_ref, zinv_ref, o_ref):
        i = pl.program_id(0)
        iz = zinv_ref[0]
        dn = (((1,), (1,)), ((), ()))
        e = jnp.exp(lax.dot_general(f_ref[...], m_ref[...], dn) * _INV_T) * iz

        col = lax.broadcasted_iota(jnp.int32, (_B, _WK), 1) + (i * _WK + kbase)
        yc = yc_ref[...]
        e = jnp.where(col == yc, jnp.exp(sv_ref[:, c0:c0 + 1] * _INV_T) * iz, e)
        e = jnp.where(col == 0, jnp.exp(sv_ref[:, cy:cy + 1] * _INV_T) * iz, e)
        o_ref[...] = e
    return _main_body


def _tc_main_half(body, mem, feat, y, sv, zinv1, half):
    ntiles = _NKT1 if half == 0 else _NKT - _NKT1
    kw = _KH1 if half == 0 else _KH2
    base = 0 if half == 0 else _NKT1
    return pl.pallas_call(
        body,
        grid=(ntiles,),
        in_specs=[
            pl.BlockSpec((_WK, _D), lambda i: (i + base, 0)),
            pl.BlockSpec((_B, _D), lambda i: (0, 0)),
            pl.BlockSpec((_B, 1), lambda i: (0, 0)),
            pl.BlockSpec((_B, 8), lambda i: (0, 0)),
            pl.BlockSpec(memory_space=pltpu.SMEM),
        ],
        out_specs=pl.BlockSpec((_B, _WK), lambda i: (0, i)),
        out_shape=jax.ShapeDtypeStruct((_B, kw), jnp.float32),
    )(mem, feat, y.reshape(_B, 1), sv, zinv1)


_main_bodies_ab = (_make_main_body(0, 2, 0), _make_main_body(0, 2, _KH1))
_main_bodies_l = (_make_main_body(1, 3, 0), _make_main_body(1, 3, _KH1))


_CH = 136
_NB = 4


def _staged_copy(srcs, dsts, lo, nrows, bufs, sems_in, sems_out):
    chunks = []
    off = 0
    while off < nrows:
        sz = min(_CH, nrows - off)
        chunks.append((off, sz))
        off += sz
    jobs = [(src, dst, o, s) for src, dst in zip(srcs, dsts)
            for (o, s) in chunks]
    n = len(jobs)
    din = [None] * n
    dout = [None] * n

    def _start_in(c):
        src, _, o, s = jobs[c]
        din[c] = pltpu.async_copy(src.at[pl.ds(lo + o, s), :],
                                  bufs[c % _NB].at[pl.ds(0, s), :],
                                  sems_in[c % _NB])

    def _start_out(c):
        _, dst, o, s = jobs[c]
        dout[c] = pltpu.async_copy(bufs[c % _NB].at[pl.ds(0, s), :],
                                   dst.at[pl.ds(lo + o, s), :],
                                   sems_out[c % _NB])

    for c in range(min(_NB, n)):
        _start_in(c)
    waited = 0
    for c in range(n):
        din[c].wait()
        _start_out(c)
        if c >= 1 and c - 1 + _NB < n:
            dout[c - 1].wait()
            waited = c
            _start_in(c - 1 + _NB)
    for c in range(waited, n):
        dout[c].wait()


def _sc_merge_body(ml_hbm, mab_hbm, ylp_hbm, posl_hbm, posab_hbm,
                   sp_hbm, cn_hbm, nml_hbm, nmab_hbm,
                   y_v, sc_v, rows_v, cbuf, semi, semo, sem):
    wid = lax.axis_index("s") * _NC + lax.axis_index("c")
    lo = wid * _SLAB
    bufs = [cbuf.at[i] for i in range(_NB)]
    sems_in = [semi.at[i] for i in range(_NB)]
    sems_out = [semo.at[i] for i in range(_NB)]

    @pl.when(wid < _NW - 1)
    def _copy_full():
        _staged_copy((ml_hbm, mab_hbm), (nml_hbm, nmab_hbm), lo, _SLAB,
                     bufs, sems_in, sems_out)

    @pl.when(wid == _NW - 1)
    def _copy_last():
        _staged_copy((ml_hbm, mab_hbm), (nml_hbm, nmab_hbm), lo, _SLAB_LAST,
                     bufs, sems_in, sems_out)

    pltpu.sync_copy(sp_hbm, sc_v.at[pl.ds(0, _NW)])
    pltpu.sync_copy(cn_hbm, sc_v.at[pl.ds(_NW, _NW)])
    pltpu.sync_copy(ylp_hbm, y_v)
    widv = jnp.full((16,), wid, jnp.int32)
    start_w = plsc.load_gather(sc_v, [widv])[0]
    count_w = plsc.load_gather(sc_v, [widv + _NW])[0]

    def _chunk(c, carry):
        off = pl.multiple_of(start_w + c * 16, 16)
        y16 = y_v[pl.ds(off, 16)]
        pltpu.sync_copy(posl_hbm.at[pl.ds(off, 16), :], rows_v)
        pltpu.async_copy(rows_v, nml_hbm.at[y16], sem).wait()
        pltpu.sync_copy(posab_hbm.at[pl.ds(off, 16), :], rows_v)
        pltpu.async_copy(rows_v, nmab_hbm.at[y16], sem).wait()
        return carry

    lax.fori_loop(0, lax.shift_right_logical(count_w + 15, 4), _chunk,
                  jnp.int32(0))


_sc_merge = pl.kernel(
    _sc_merge_body,
    out_type=(jax.ShapeDtypeStruct((_K, _D), jnp.float32),
              jax.ShapeDtypeStruct((_K, _D), jnp.float32)),
    mesh=plsc.VectorSubcoreMesh(core_axis_name="c", subcore_axis_name="s",
                                num_cores=_NC, num_subcores=_NS),
    scratch_types=[pltpu.VMEM((_PAD,), jnp.int32),
                   pltpu.VMEM((2 * _NW,), jnp.int32),
                   pltpu.VMEM((16, _D), jnp.float32),
                   pltpu.VMEM((_NB, _CH, _D), jnp.float32),
                   pltpu.SemaphoreType.DMA((_NB,)),
                   pltpu.SemaphoreType.DMA((_NB,)),
                   pltpu.SemaphoreType.DMA],
    compiler_params=pltpu.CompilerParams(needs_layout_passes=False),
)


def kernel(l, ab, y, memory_l, memory_ab):
    gl, gab = _sc_gather(memory_l, memory_ab, y)
    posl, posab, sv, ylp, startp, count = _tc_prep(
        l, ab, gl, gab, y, memory_l[0:1, :], memory_ab[0:1, :])
    sums = _tc_z(memory_l, memory_ab, l, ab)
    zinv = jnp.float32(_B) / sums
    nml, nmab = _sc_merge(memory_l, memory_ab, ylp.reshape(_PAD), posl, posab,
                          startp.reshape(_NW), count.reshape(_NW))

    out_l_h1 = _tc_main_half(_main_bodies_l[0], memory_ab, l, y, sv,
                             zinv[1:2], 0)
    mab_b, l_b, y_b, sv_b, zl_b, _, nml_b, nmab_b = lax.optimization_barrier(
        (memory_ab, l, y, sv, zinv[1:2], out_l_h1, nml, nmab))
    out_l_h2 = _tc_main_half(_main_bodies_l[1], mab_b, l_b, y_b, sv_b,
                             zl_b, 1)
    ml_c, ab_c, y_c, sv_c, zab_c, _ = lax.optimization_barrier(
        (memory_l, ab, y, sv, zinv[0:1], out_l_h2))
    out_ab_h1 = _tc_main_half(_main_bodies_ab[0], ml_c, ab_c, y_c, sv_c,
                              zab_c, 0)
    ml_d, ab_d, y_d, sv_d, zab_d, _ = lax.optimization_barrier(
        (memory_l, ab, y, sv, zinv[0:1], out_ab_h1))
    out_ab_h2 = _tc_main_half(_main_bodies_ab[1], ml_d, ab_d, y_d, sv_d,
                              zab_d, 1)
    out_l2 = jnp.concatenate([out_l_h1, out_l_h2], axis=1)
    out_ab2 = jnp.concatenate([out_ab_h1, out_ab_h2], axis=1)
    return (out_l2.reshape(_B, _K, 1), out_ab2.reshape(_B, _K, 1),
            nml_b, nmab_b)


# speedup vs baseline: 1.4303x; 1.4303x over previous
import functools
import jax
import jax.numpy as jnp
from jax import lax
from jax.experimental import pallas as pl
from jax.experimental.pallas import tpu as pltpu
from jax.experimental.pallas import tpu_sc as plsc

_B = 1024
_K = 100000
_D = 128
_T = 0.07
_INV_T = 1.0 / _T

_NC = 2
_NS = 16
_NW = _NC * _NS
_BPW = _B // _NW
_SLAB = 3128
_SLAB_LAST = _K - _SLAB * (_NW - 1)
_PAD = _B + _NW * 16

_WK = 2048
_NKT = (_K + _WK - 1) // _WK


def _sc_gather_body(ml_hbm, mab_hbm, y_hbm, gl_hbm, gab_hbm, idx_v, rows_v, sem):
    wid = lax.axis_index("s") * _NC + lax.axis_index("c")
    base = wid * _BPW
    pltpu.sync_copy(y_hbm.at[pl.ds(base, _BPW)], idx_v)
    pltpu.async_copy(ml_hbm.at[idx_v], rows_v, sem).wait()
    pltpu.sync_copy(rows_v, gl_hbm.at[pl.ds(base, _BPW)])
    pltpu.async_copy(mab_hbm.at[idx_v], rows_v, sem).wait()
    pltpu.sync_copy(rows_v, gab_hbm.at[pl.ds(base, _BPW)])


_sc_gather = pl.kernel(
    _sc_gather_body,
    out_type=(jax.ShapeDtypeStruct((_B, _D), jnp.float32),
              jax.ShapeDtypeStruct((_B, _D), jnp.float32)),
    mesh=plsc.VectorSubcoreMesh(core_axis_name="c", subcore_axis_name="s",
                                num_cores=_NC, num_subcores=_NS),
    scratch_types=[pltpu.VMEM((_BPW,), jnp.int32),
                   pltpu.VMEM((_BPW, _D), jnp.float32),
                   pltpu.SemaphoreType.DMA],
    compiler_params=pltpu.CompilerParams(needs_layout_passes=False),
)


def _prep_body(l_ref, ab_ref, gl_ref, gab_ref, yr_ref, yc_ref, ml0_ref, mab0_ref,
               posl_ref, posab_ref, sv_ref, ylp_ref, startp_ref, count_ref):
    yr = yr_ref[...]
    yc = yc_ref[...]
    eq = yc == yr
    rowb = lax.broadcasted_iota(jnp.int32, (_B, _B), 0)
    colb = lax.broadcasted_iota(jnp.int32, (_B, _B), 1)
    winner = jnp.max(jnp.where(eq, colb, -1), axis=1, keepdims=True)
    onehot = (colb == winner).astype(jnp.float32)

    hp = lax.Precision.HIGHEST
    lw = jnp.dot(onehot, l_ref[...], precision=hp)
    abw = jnp.dot(onehot, ab_ref[...], precision=hp)
    glw = jnp.dot(onehot, gl_ref[...], precision=hp)
    gabw = jnp.dot(onehot, gab_ref[...], precision=hp)

    posl = glw * 0.5 + lw * 0.5
    posl = posl / jnp.sqrt(jnp.sum(posl * posl, axis=1, keepdims=True))
    posab = gabw * 0.5 + abw * 0.5
    posab = posab / jnp.sqrt(jnp.sum(posab * posab, axis=1, keepdims=True))

    v_y_ab = jnp.sum(ab_ref[...] * gl_ref[...], axis=1, keepdims=True)
    v_y_l = jnp.sum(l_ref[...] * gab_ref[...], axis=1, keepdims=True)
    v_0_ab = jnp.sum(ab_ref[...] * ml0_ref[...], axis=1, keepdims=True)
    v_0_l = jnp.sum(l_ref[...] * mab0_ref[...], axis=1, keepdims=True)
    sv_ref[...] = jnp.concatenate(
        [v_y_ab, v_y_l, v_0_ab, v_0_l, v_y_ab, v_y_l, v_0_ab, v_0_l], axis=1)

    tbound_row = lax.broadcasted_iota(jnp.int32, (1, _NW), 1) * _SLAB
    tbound_col = lax.broadcasted_iota(jnp.int32, (_NW, 1), 0) * _SLAB
    tile_col = jnp.sum((yc >= tbound_row).astype(jnp.int32),
                       axis=1, keepdims=True) - 1
    tile_row = jnp.sum((yr >= tbound_col).astype(jnp.int32),
                       axis=0, keepdims=True) - 1
    teq = tile_col == tile_row
    rank = jnp.sum(jnp.where(teq & (colb < rowb), 1, 0),
                   axis=1, keepdims=True)

    onehot_bt = (tile_col == lax.broadcasted_iota(jnp.int32, (1, _NW), 1))
    onehot_bt = onehot_bt.astype(jnp.float32)
    dn0 = (((0,), (0,)), ((), ()))
    ones_b = jnp.ones((_B, 1), jnp.float32)
    count = lax.dot_general(onehot_bt, ones_b, dn0, precision=hp)
    count_i = count.astype(jnp.int32)
    countp_i = lax.shift_left(lax.shift_right_logical(count_i + 15, 2 + 2), 4)
    countp = countp_i.astype(jnp.float32)
    lt32 = (lax.broadcasted_iota(jnp.int32, (_NW, _NW), 1)
            < lax.broadcasted_iota(jnp.int32, (_NW, _NW), 0)).astype(jnp.float32)
    startp = jnp.dot(lt32, countp, precision=hp)

    startp_b = jnp.dot(onehot_bt, startp, precision=hp).astype(jnp.int32)
    count_b = jnp.dot(onehot_bt, count, precision=hp).astype(jnp.int32)
    countp_b = jnp.dot(onehot_bt, countp, precision=hp).astype(jnp.int32)
    posidx = startp_b + rank

    p_row = lax.broadcasted_iota(jnp.int32, (_B, _PAD), 1)
    padmask = ((rank == 0) & (p_row >= startp_b + count_b)
               & (p_row < startp_b + countp_b))
    pt = ((p_row == posidx) | padmask).astype(jnp.float32)

    posl_ref[...] = lax.dot_general(pt, posl, dn0, precision=hp)
    posab_ref[...] = lax.dot_general(pt, posab, dn0, precision=hp)
    ylp_ref[...] = lax.dot_general(pt, yc.astype(jnp.float32), dn0,
                                   precision=hp).astype(jnp.int32)
    startp_ref[...] = startp.astype(jnp.int32)
    count_ref[...] = count_i


def _tc_prep(l, ab, gl, gab, y, ml0, mab0):
    return pl.pallas_call(
        _prep_body,
        out_shape=(jax.ShapeDtypeStruct((_PAD, _D), jnp.float32),
                   jax.ShapeDtypeStruct((_PAD, _D), jnp.float32),
                   jax.ShapeDtypeStruct((_B, 8), jnp.float32),
                   jax.ShapeDtypeStruct((_PAD, 1), jnp.int32),
                   jax.ShapeDtypeStruct((_NW, 1), jnp.int32),
                   jax.ShapeDtypeStruct((_NW, 1), jnp.int32)),
    )(l, ab, gl, gab, y.reshape(1, _B), y.reshape(_B, 1), ml0, mab0)


def _z_body(ml_ref, mab_ref, l_ref, ab_ref, out_ref):
    i = pl.program_id(0)

    @pl.when(i == 0)
    def _init():
        out_ref[0] = 0.0
        out_ref[1] = 0.0

    col = lax.broadcasted_iota(jnp.int32, (_B, _WK), 1) + i * _WK
    valid = col < _K
    dn = (((1,), (1,)), ((), ()))
    s_ab = lax.dot_general(ab_ref[...], ml_ref[...], dn) * _INV_T
    s_l = lax.dot_general(l_ref[...], mab_ref[...], dn) * _INV_T
    out_ref[0] += jnp.sum(jnp.where(valid, jnp.exp(s_ab), 0.0))
    out_ref[1] += jnp.sum(jnp.where(valid, jnp.exp(s_l), 0.0))


def _tc_z(memory_l, memory_ab, l, ab):
    return pl.pallas_call(
        _z_body,
        grid=(_NKT,),
        in_specs=[
            pl.BlockSpec((_WK, _D), lambda i: (i, 0)),
            pl.BlockSpec((_WK, _D), lambda i: (i, 0)),
            pl.BlockSpec((_B, _D), lambda i: (0, 0)),
            pl.BlockSpec((_B, _D), lambda i: (0, 0)),
        ],
        out_specs=pl.BlockSpec(memory_space=pltpu.SMEM),
        out_shape=jax.ShapeDtypeStruct((2,), jnp.float32),
    )(memory_l, memory_ab, l, ab)


def _make_main_body(cy, c0):
    def _main_body(m_ref, f_ref, yc_ref, sv_ref, zinv_ref, o_ref):
        i = pl.program_id(0)
        iz = zinv_ref[0]
        dn = (((1,), (1,)), ((), ()))
        e = jnp.exp(lax.dot_general(f_ref[...], m_ref[...], dn) * _INV_T) * iz

        col = lax.broadcasted_iota(jnp.int32, (_B, _WK), 1) + i * _WK
        yc = yc_ref[...]
        e = jnp.where(col == yc, jnp.exp(sv_ref[:, c0:c0 + 1] * _INV_T) * iz, e)
        e = jnp.where(col == 0, jnp.exp(sv_ref[:, cy:cy + 1] * _INV_T) * iz, e)
        o_ref[...] = e
    return _main_body


def _tc_main_one(body, mem, feat, y, sv, zinv1):
    return pl.pallas_call(
        body,
        grid=(_NKT,),
        in_specs=[
            pl.BlockSpec((_WK, _D), lambda i: (i, 0)),
            pl.BlockSpec((_B, _D), lambda i: (0, 0)),
            pl.BlockSpec((_B, 1), lambda i: (0, 0)),
            pl.BlockSpec((_B, 8), lambda i: (0, 0)),
            pl.BlockSpec(memory_space=pltpu.SMEM),
        ],
        out_specs=pl.BlockSpec((_B, _WK), lambda i: (0, i)),
        out_shape=jax.ShapeDtypeStruct((_B, _K), jnp.float32),
    )(mem, feat, y.reshape(_B, 1), sv, zinv1)


_main_body_ab = _make_main_body(0, 2)
_main_body_l = _make_main_body(1, 3)


_CH = 136
_NB = 4


def _staged_copy(srcs, dsts, lo, nrows, bufs, sems_in, sems_out):
    chunks = []
    off = 0
    while off < nrows:
        sz = min(_CH, nrows - off)
        chunks.append((off, sz))
        off += sz
    jobs = [(src, dst, o, s) for src, dst in zip(srcs, dsts)
            for (o, s) in chunks]
    n = len(jobs)
    din = [None] * n
    dout = [None] * n

    def _start_in(c):
        src, _, o, s = jobs[c]
        din[c] = pltpu.async_copy(src.at[pl.ds(lo + o, s), :],
                                  bufs[c % _NB].at[pl.ds(0, s), :],
                                  sems_in[c % _NB])

    def _start_out(c):
        _, dst, o, s = jobs[c]
        dout[c] = pltpu.async_copy(bufs[c % _NB].at[pl.ds(0, s), :],
                                   dst.at[pl.ds(lo + o, s), :],
                                   sems_out[c % _NB])

    for c in range(min(_NB, n)):
        _start_in(c)
    waited = 0
    for c in range(n):
        din[c].wait()
        _start_out(c)
        if c >= 1 and c - 1 + _NB < n:
            dout[c - 1].wait()
            waited = c
            _start_in(c - 1 + _NB)
    for c in range(waited, n):
        dout[c].wait()


def _sc_merge_body(ml_hbm, mab_hbm, ylp_hbm, posl_hbm, posab_hbm,
                   sp_hbm, cn_hbm, nml_hbm, nmab_hbm,
                   y_v, sc_v, rows_v, cbuf, semi, semo, sem):
    wid = lax.axis_index("s") * _NC + lax.axis_index("c")
    lo = wid * _SLAB
    bufs = [cbuf.at[i] for i in range(_NB)]
    sems_in = [semi.at[i] for i in range(_NB)]
    sems_out = [semo.at[i] for i in range(_NB)]

    @pl.when(wid < _NW - 1)
    def _copy_full():
        _staged_copy((ml_hbm, mab_hbm), (nml_hbm, nmab_hbm), lo, _SLAB,
                     bufs, sems_in, sems_out)

    @pl.when(wid == _NW - 1)
    def _copy_last():
        _staged_copy((ml_hbm, mab_hbm), (nml_hbm, nmab_hbm), lo, _SLAB_LAST,
                     bufs, sems_in, sems_out)

    pltpu.sync_copy(sp_hbm, sc_v.at[pl.ds(0, _NW)])
    pltpu.sync_copy(cn_hbm, sc_v.at[pl.ds(_NW, _NW)])
    pltpu.sync_copy(ylp_hbm, y_v)
    widv = jnp.full((16,), wid, jnp.int32)
    start_w = plsc.load_gather(sc_v, [widv])[0]
    count_w = plsc.load_gather(sc_v, [widv + _NW])[0]

    def _chunk(c, carry):
        off = pl.multiple_of(start_w + c * 16, 16)
        y16 = y_v[pl.ds(off, 16)]
        pltpu.sync_copy(posl_hbm.at[pl.ds(off, 16), :], rows_v)
        pltpu.async_copy(rows_v, nml_hbm.at[y16], sem).wait()
        pltpu.sync_copy(posab_hbm.at[pl.ds(off, 16), :], rows_v)
        pltpu.async_copy(rows_v, nmab_hbm.at[y16], sem).wait()
        return carry

    lax.fori_loop(0, lax.shift_right_logical(count_w + 15, 4), _chunk,
                  jnp.int32(0))


_sc_merge = pl.kernel(
    _sc_merge_body,
    out_type=(jax.ShapeDtypeStruct((_K, _D), jnp.float32),
              jax.ShapeDtypeStruct((_K, _D), jnp.float32)),
    mesh=plsc.VectorSubcoreMesh(core_axis_name="c", subcore_axis_name="s",
                                num_cores=_NC, num_subcores=_NS),
    scratch_types=[pltpu.VMEM((_PAD,), jnp.int32),
                   pltpu.VMEM((2 * _NW,), jnp.int32),
                   pltpu.VMEM((16, _D), jnp.float32),
                   pltpu.VMEM((_NB, _CH, _D), jnp.float32),
                   pltpu.SemaphoreType.DMA((_NB,)),
                   pltpu.SemaphoreType.DMA((_NB,)),
                   pltpu.SemaphoreType.DMA],
    compiler_params=pltpu.CompilerParams(needs_layout_passes=False),
)


def kernel(l, ab, y, memory_l, memory_ab):
    gl, gab = _sc_gather(memory_l, memory_ab, y)
    posl, posab, sv, ylp, startp, count = _tc_prep(
        l, ab, gl, gab, y, memory_l[0:1, :], memory_ab[0:1, :])
    sums = _tc_z(memory_l, memory_ab, l, ab)
    zinv = jnp.float32(_B) / sums
    nml, nmab = _sc_merge(memory_l, memory_ab, ylp.reshape(_PAD), posl, posab,
                          startp.reshape(_NW), count.reshape(_NW))

    out_l2 = _tc_main_one(_main_body_l, memory_ab, l, y, sv, zinv[1:2])
    ml_b, ab_b, y_b, sv_b, zab_b, _, nml_b, nmab_b = lax.optimization_barrier(
        (memory_l, ab, y, sv, zinv[0:1], out_l2, nml, nmab))
    out_ab2 = _tc_main_one(_main_body_ab, ml_b, ab_b, y_b, sv_b, zab_b)
    return (out_l2.reshape(_B, _K, 1), out_ab2.reshape(_B, _K, 1),
            nml_b, nmab_b)
